# baseline (speedup 1.0000x reference)
import functools

import jax
import jax.numpy as jnp
from jax import lax
from jax.experimental import pallas as pl
from jax.experimental.pallas import tpu as pltpu
from jax.experimental.pallas import tpu_sc as plsc

BATCH = 16384
FEAT = 32
NC = 2
NS = 16
NW = NC * NS
BPW = BATCH // NW
LANES = 16


def _body(idx_hbm, cb0_hbm, cb1_hbm, cb2_hbm, cb3_hbm, out_hbm,
          idx_v, i0_v, i1_v, i2_v, r0, r1, r2, r3, sem):
    wid = lax.axis_index("s") * NC + lax.axis_index("c")
    base = wid * BPW

    pltpu.sync_copy(idx_hbm.at[pl.ds(base, BPW)], idx_v)

    def mask_body(j, _):
        s = pl.ds(j * LANES, LANES)
        v = idx_v[s]
        i0_v[s] = lax.bitwise_and(v, 4095)
        i1_v[s] = lax.bitwise_and(v, 16383)
        i2_v[s] = lax.bitwise_and(v, 65535)
        return 0

    lax.fori_loop(0, BPW // LANES, mask_body, 0, unroll=2)

    c0 = pltpu.async_copy(cb0_hbm.at[i0_v], r0, sem)
    c1 = pltpu.async_copy(cb1_hbm.at[i1_v], r1, sem)
    c2 = pltpu.async_copy(cb2_hbm.at[i2_v], r2, sem)
    c3 = pltpu.async_copy(cb3_hbm.at[idx_v], r3, sem)
    c0.wait()
    c1.wait()
    c2.wait()
    c3.wait()

    def add_body(i, _):
        for u in range(2):
            row = i * 2 + u
            for h in range(2):
                s = pl.ds(h * LANES, LANES)
                a = r0[row, s] + r1[row, s]
                b = r2[row, s] + r3[row, s]
                r0[row, s] = a + b
        return 0

    lax.fori_loop(0, BPW // 2, add_body, 0, unroll=2)

    pltpu.sync_copy(r0, out_hbm.at[pl.ds(base, BPW)])


@jax.jit
def _octree_lookup(indices, cb0, cb1, cb2, cb3):
    mesh = plsc.VectorSubcoreMesh(core_axis_name="c", subcore_axis_name="s")
    f = functools.partial(
        pl.kernel,
        mesh=mesh,
        compiler_params=pltpu.CompilerParams(use_tc_tiling_on_sc=False),
        out_type=jax.ShapeDtypeStruct((BATCH, FEAT), jnp.float32),
        scratch_types=[
            pltpu.VMEM((BPW,), jnp.int32),
            pltpu.VMEM((BPW,), jnp.int32),
            pltpu.VMEM((BPW,), jnp.int32),
            pltpu.VMEM((BPW,), jnp.int32),
            pltpu.VMEM((BPW, FEAT), jnp.float32),
            pltpu.VMEM((BPW, FEAT), jnp.float32),
            pltpu.VMEM((BPW, FEAT), jnp.float32),
            pltpu.VMEM((BPW, FEAT), jnp.float32),
            pltpu.SemaphoreType.DMA,
        ],
    )(_body)
    return f(indices, cb0, cb1, cb2, cb3)


def kernel(indices, cb0, cb1, cb2, cb3):
    return _octree_lookup(indices.astype(jnp.int32), cb0, cb1, cb2, cb3)
